# Initial kernel scaffold; baseline (speedup 1.0000x reference)
import jax
import jax.numpy as jnp
from jax.experimental import pallas as pl


def kernel(x, annotators, questions, embeddings, params):
    raise NotImplementedError("write your pallas kernel here")


# speedup vs baseline: 1.9634x; 1.9634x over previous
import functools
import math

import jax
import jax.numpy as jnp
from jax import lax
from jax.experimental import pallas as pl
from jax.experimental.pallas import tpu as pltpu
from jax.experimental.pallas import tpu_sc as plsc

QN = 20
MC = 8
L = 4
H = 4
NA = 1000
AED = 32
F = AED + MC + 384
P = MC
DFF = 4 * F
DH = F // H

FP = 512
DHP = 128
DFFP = 1792
HLF = F // 2
HLFP = 256


def _sc_gather(q_emb, a_emb, qidx, aidx):
    n = qidx.shape[0]
    d = q_emb.shape[1]
    info = plsc.get_sparse_core_info()
    nc, ns = info.num_cores, info.num_subcores
    nw = nc * ns
    per_w = n // nw
    mesh = plsc.VectorSubcoreMesh(core_axis_name="c", subcore_axis_name="s")

    @functools.partial(
        pl.kernel,
        mesh=mesh,
        out_type=[
            jax.ShapeDtypeStruct((n, d), jnp.float32),
            jax.ShapeDtypeStruct((n, d), jnp.float32),
        ],
        scratch_types=[
            pltpu.VMEM((per_w,), jnp.int32),
            pltpu.VMEM((per_w,), jnp.int32),
            pltpu.VMEM((per_w, d), jnp.float32),
            pltpu.VMEM((per_w, d), jnp.float32),
            pltpu.SemaphoreType.DMA,
            pltpu.SemaphoreType.DMA,
        ],
    )
    def gather_k(qt_hbm, at_hbm, qi_hbm, ai_hbm, qo_hbm, ao_hbm,
                 qi_v, ai_v, qr_v, ar_v, sem_q, sem_a):
        wid = lax.axis_index("s") * nc + lax.axis_index("c")
        base = wid * per_w
        pltpu.sync_copy(qi_hbm.at[pl.ds(base, per_w)], qi_v)
        pltpu.sync_copy(ai_hbm.at[pl.ds(base, per_w)], ai_v)
        cq = pltpu.async_copy(qt_hbm.at[qi_v], qr_v, sem_q)
        ca = pltpu.async_copy(at_hbm.at[ai_v], ar_v, sem_a)
        cq.wait()
        ca.wait()
        pltpu.sync_copy(qr_v, qo_hbm.at[pl.ds(base, per_w)])
        pltpu.sync_copy(ar_v, ao_hbm.at[pl.ds(base, per_w)])

    return gather_k(q_emb, a_emb, qidx, aidx)


def _ln(y, a, b, fmask):
    m = jnp.sum(y, axis=-1, keepdims=True) * (1.0 / F)
    c = y - m
    var = jnp.sum(c * c * fmask, axis=-1, keepdims=True) * (1.0 / (F - 1))
    return a * (c / (jnp.sqrt(var) + 1e-6)) + b


def _layer_body(fx_ref, px_ref, qrow_ref, qcol_ref, fmask_ref,
                qw, qb, kw, kb, vw, vb, ow, ob,
                f1w, f1b, f2w, f2b,
                n1a, n1b, n2a, n2b,
                pufw, pupw, pub,
                s1w, s1b, s2w, s2b,
                c1w, c1b, c2w, c2b,
                fx_out, px_out):
    fx = fx_ref[0]
    px = px_ref[0]
    fmask = fmask_ref[...]

    q = jnp.dot(fx, qw[...], preferred_element_type=jnp.float32) + qb[...]
    k = jnp.dot(fx, kw[...], preferred_element_type=jnp.float32) + kb[...]
    v = jnp.dot(fx, vw[...], preferred_element_type=jnp.float32) + vb[...]

    scale = 1.0 / math.sqrt(DH)
    ctxs = []
    for h in range(H):
        sl = slice(h * DHP, (h + 1) * DHP)
        qh, kh, vh = q[:, sl], k[:, sl], v[:, sl]
        sc = lax.dot_general(qh, kh, (((1,), (1,)), ((), ())),
                             preferred_element_type=jnp.float32) * scale
        sc = sc - jnp.max(sc, axis=-1, keepdims=True)
        e = jnp.exp(sc)
        p = e / jnp.sum(e, axis=-1, keepdims=True)
        ctxs.append(jnp.dot(p, vh, preferred_element_type=jnp.float32))
    ctx = jnp.concatenate(ctxs, axis=-1)
    att = jnp.dot(ctx, ow[...], preferred_element_type=jnp.float32) + ob[...]

    fx1 = _ln(fx + att, n1a[...], n1b[...], fmask)

    ff = jnp.maximum(
        jnp.dot(fx1, f1w[...], preferred_element_type=jnp.float32) + f1b[...],
        0.0)
    ff = jnp.dot(ff, f2w[...], preferred_element_type=jnp.float32) + f2b[...]
    fx2 = _ln(fx1 + ff, n2a[...], n2b[...], fmask)

    px_new = (jnp.dot(fx2, pufw[...], preferred_element_type=jnp.float32)
              + jnp.dot(px, pupw[...], preferred_element_type=jnp.float32)
              + pub[...])

    h1 = jnp.maximum(
        jnp.dot(fx2, s1w[...], preferred_element_type=jnp.float32) + s1b[...],
        0.0)
    sim = jnp.sum(h1 * s2w[...], axis=-1, keepdims=True) + s2b[...]
    h2 = jnp.maximum(
        jnp.dot(fx2, c1w[...], preferred_element_type=jnp.float32) + c1b[...],
        0.0)
    conf = jax.nn.sigmoid(
        jnp.sum(h2 * c2w[...], axis=-1, keepdims=True) + c2b[...])

    qrow = qrow_ref[0]
    qcol = qcol_ref[0]
    qmask = (qcol == qrow).astype(jnp.float32)
    m = sim * qmask
    m = m - jnp.max(m, axis=0, keepdims=True)
    e = jnp.exp(m)
    aw = e / jnp.sum(e, axis=0, keepdims=True)
    smoothed = lax.dot_general(aw, px_new, (((0,), (0,)), ((), ())),
                               preferred_element_type=jnp.float32)

    fx_out[0] = fx2
    px_out[0] = conf * px_new + (1.0 - conf) * smoothed


def _pad2(w, rows, cols):
    return jnp.pad(w, ((0, rows - w.shape[0]), (0, cols - w.shape[1])))


def _perm_cols(w):
    chunks = [jnp.pad(w[:, h * DH:(h + 1) * DH], ((0, 0), (0, DHP - DH)))
              for h in range(H)]
    return jnp.concatenate(chunks, axis=1)


def _perm_rows(w):
    chunks = [jnp.pad(w[h * DH:(h + 1) * DH, :], ((0, DHP - DH), (0, 0)))
              for h in range(H)]
    return jnp.concatenate(chunks, axis=0)


def _pad_vec(b, n):
    return jnp.pad(b, (0, n - b.shape[0])).reshape(1, n)


def _perm_vec(b):
    chunks = [jnp.pad(b[h * DH:(h + 1) * DH], (0, DHP - DH)) for h in range(H)]
    return jnp.concatenate(chunks).reshape(1, FP)


def _prep_layer(lp):
    return (
        _pad2(_perm_cols(lp["Qw"]), FP, FP), _perm_vec(lp["Qb"]),
        _pad2(_perm_cols(lp["Kw"]), FP, FP), _perm_vec(lp["Kb"]),
        _pad2(_perm_cols(lp["Vw"]), FP, FP), _perm_vec(lp["Vb"]),
        _pad2(_perm_rows(lp["Ow"]), FP, FP), _pad_vec(lp["Ob"], FP),
        _pad2(lp["ff1w"], FP, DFFP), _pad_vec(lp["ff1b"], DFFP),
        _pad2(lp["ff2w"], DFFP, FP), _pad_vec(lp["ff2b"], FP),
        _pad_vec(lp["n1a"], FP), _pad_vec(lp["n1b"], FP),
        _pad_vec(lp["n2a"], FP), _pad_vec(lp["n2b"], FP),
        _pad2(lp["puw"][:F], FP, P), lp["puw"][F:], lp["pub"].reshape(1, P),
        _pad2(lp["s1w"], FP, HLFP), _pad_vec(lp["s1b"], HLFP),
        _pad_vec(lp["s2w"][:, 0], HLFP), lp["s2b"].reshape(1, 1),
        _pad2(lp["c1w"], FP, HLFP), _pad_vec(lp["c1b"], HLFP),
        _pad_vec(lp["c2w"][:, 0], HLFP), lp["c2b"].reshape(1, 1),
    )


def _whole(shape):
    nd = len(shape)
    return pl.BlockSpec(shape, lambda b, _nd=nd: (0,) * _nd)


def _layer_call(fx, px, qrow, qcol, fmask, wts, interpret=False):
    b, s, _ = fx.shape
    in_specs = [
        pl.BlockSpec((1, s, FP), lambda i: (i, 0, 0)),
        pl.BlockSpec((1, s, P), lambda i: (i, 0, 0)),
        pl.BlockSpec((1, 1, s), lambda i: (i, 0, 0)),
        pl.BlockSpec((1, s, 1), lambda i: (i, 0, 0)),
        _whole(fmask.shape),
    ] + [_whole(w.shape) for w in wts]
    out_specs = [
        pl.BlockSpec((1, s, FP), lambda i: (i, 0, 0)),
        pl.BlockSpec((1, s, P), lambda i: (i, 0, 0)),
    ]
    return pl.pallas_call(
        _layer_body,
        grid=(b,),
        in_specs=in_specs,
        out_specs=out_specs,
        out_shape=[
            jax.ShapeDtypeStruct((b, s, FP), jnp.float32),
            jax.ShapeDtypeStruct((b, s, P), jnp.float32),
        ],
        compiler_params=pltpu.CompilerParams(
            dimension_semantics=("arbitrary",),
        ),
        interpret=interpret,
    )(fx, px, qrow, qcol, fmask, *wts)


def kernel(x, annotators, questions, embeddings, params):
    b, s = annotators.shape
    qidx = questions.astype(jnp.int32)
    ann = annotators.astype(jnp.int32)
    aidx = jnp.where(ann < 0, NA, ann)

    qt = jnp.pad(params["q_emb"], ((0, 0), (0, 128 - AED)))
    at = jnp.pad(params["a_emb"], ((0, 0), (0, 128 - AED)))
    qe, ae = _sc_gather(qt, at, qidx.reshape(-1), aidx.reshape(-1))
    emb = (qe + ae)[:, :AED].reshape(b, s, AED)

    fx = jnp.concatenate(
        [emb, embeddings, x[:, :, 1:],
         jnp.zeros((b, s, FP - F), jnp.float32)], axis=-1)
    px = x[:, :, 1:]
    qrow = qidx.reshape(b, 1, s)
    qcol = qidx.reshape(b, s, 1)
    fmask = (jnp.arange(FP) < F).astype(jnp.float32).reshape(1, FP)

    for lp in params["layers"]:
        fx, px = _layer_call(fx, px, qrow, qcol, fmask, _prep_layer(lp))
    return px


# speedup vs baseline: 2.4155x; 1.2302x over previous
import functools
import math

import jax
import jax.numpy as jnp
from jax import lax
from jax.experimental import pallas as pl
from jax.experimental.pallas import tpu as pltpu
from jax.experimental.pallas import tpu_sc as plsc

QN = 20
MC = 8
NLAYER = 4
H = 4
NA = 1000
AED = 32
F = AED + MC + 384
P = MC
DFF = 4 * F
DH = F // H

FP = 512
DHP = 128
DFFP = 1792
HLF = F // 2
HLFP = 256

BB = 2


def _sc_gather(q_emb, a_emb, qidx, aidx):
    n = qidx.shape[0]
    d = q_emb.shape[1]
    info = plsc.get_sparse_core_info()
    nc, ns = info.num_cores, info.num_subcores
    nw = nc * ns
    per_w = n // nw
    mesh = plsc.VectorSubcoreMesh(core_axis_name="c", subcore_axis_name="s")

    @functools.partial(
        pl.kernel,
        mesh=mesh,
        out_type=[
            jax.ShapeDtypeStruct((n, d), jnp.float32),
            jax.ShapeDtypeStruct((n, d), jnp.float32),
        ],
        scratch_types=[
            pltpu.VMEM((per_w,), jnp.int32),
            pltpu.VMEM((per_w,), jnp.int32),
            pltpu.VMEM((per_w, d), jnp.float32),
            pltpu.VMEM((per_w, d), jnp.float32),
            pltpu.SemaphoreType.DMA,
            pltpu.SemaphoreType.DMA,
        ],
    )
    def gather_k(qt_hbm, at_hbm, qi_hbm, ai_hbm, qo_hbm, ao_hbm,
                 qi_v, ai_v, qr_v, ar_v, sem_q, sem_a):
        wid = lax.axis_index("s") * nc + lax.axis_index("c")
        base = wid * per_w
        pltpu.sync_copy(qi_hbm.at[pl.ds(base, per_w)], qi_v)
        pltpu.sync_copy(ai_hbm.at[pl.ds(base, per_w)], ai_v)
        cq = pltpu.async_copy(qt_hbm.at[qi_v], qr_v, sem_q)
        ca = pltpu.async_copy(at_hbm.at[ai_v], ar_v, sem_a)
        cq.wait()
        ca.wait()
        pltpu.sync_copy(qr_v, qo_hbm.at[pl.ds(base, per_w)])
        pltpu.sync_copy(ar_v, ao_hbm.at[pl.ds(base, per_w)])

    return gather_k(q_emb, a_emb, qidx, aidx)


RAW_KEYS = ("Qw", "Qb", "Kw", "Kb", "Vw", "Vb", "Ow", "Ob",
            "ff1w", "ff1b", "ff2w", "ff2b",
            "n1a", "n1b", "n2a", "n2b",
            "puw", "pub",
            "s1w", "s1b", "s2w", "s2b",
            "c1w", "c1b", "c2w", "c2b")


def _bf(x):
    return x.astype(jnp.bfloat16)


def _pad_rc(w, rows, cols):
    r, c = w.shape
    if cols > c:
        w = jnp.concatenate([w, jnp.zeros((r, cols - c), w.dtype)], axis=1)
    if rows > r:
        w = jnp.concatenate([w, jnp.zeros((rows - r, cols), w.dtype)], axis=0)
    return w


def _perm_cols(w):
    z = jnp.zeros((w.shape[0], DHP - DH), w.dtype)
    parts = []
    for h in range(H):
        parts.append(w[:, h * DH:(h + 1) * DH])
        parts.append(z)
    return jnp.concatenate(parts, axis=1)


def _perm_rows(w):
    z = jnp.zeros((DHP - DH, w.shape[1]), w.dtype)
    parts = []
    for h in range(H):
        parts.append(w[h * DH:(h + 1) * DH, :])
        parts.append(z)
    return jnp.concatenate(parts, axis=0)


def _ln(y, a, b, fmask):
    m = jnp.sum(y, axis=-1, keepdims=True) * (1.0 / F)
    c = y - m
    var = jnp.sum(c * c * fmask, axis=-1, keepdims=True) * (1.0 / (F - 1))
    return a * (c / (jnp.sqrt(var) + 1e-6)) + b


def _layer_body(args, write_fx):
    (fx_ref, px_ref, qrow_ref, qcol_ref,
     rQw, rQb, rKw, rKb, rVw, rVb, rOw, rOb,
     rf1w, rf1b, rf2w, rf2b,
     rn1a, rn1b, rn2a, rn2b,
     rpuw, rpub,
     rs1w, rs1b, rs2w, rs2b,
     rc1w, rc1b, rc2w, rc2b) = args[:30]
    outs = args[30:30 + (2 if write_fx else 1)]
    (qw_s, qb_s, kw_s, kb_s, vw_s, vb_s, ow_s, ob_s,
     f1w_s, f1b_s, f2w_s, f2b_s,
     n1a_s, n1b_s, n2a_s, n2b_s,
     pf_s, pp_s,
     s1w_s, s1b_s, s2w_s,
     c1w_s, c1b_s, c2w_s) = args[30 + len(outs):]
    if write_fx:
        fx_out, px_out = outs
    else:
        fx_out, (px_out,) = None, outs

    @pl.when(pl.program_id(0) == 0)
    def _prep():
        qw_s[...] = _bf(_pad_rc(_perm_cols(rQw[...]), FP, FP))
        kw_s[...] = _bf(_pad_rc(_perm_cols(rKw[...]), FP, FP))
        vw_s[...] = _bf(_pad_rc(_perm_cols(rVw[...]), FP, FP))
        ow_s[...] = _bf(_pad_rc(_perm_rows(rOw[...]), FP, FP))
        qb_s[...] = _perm_cols(rQb[...].reshape(1, F))
        kb_s[...] = _perm_cols(rKb[...].reshape(1, F))
        vb_s[...] = _perm_cols(rVb[...].reshape(1, F))
        ob_s[...] = _pad_rc(rOb[...].reshape(1, F), 1, FP)
        f1w_s[...] = _bf(_pad_rc(rf1w[...], FP, DFFP))
        f1b_s[...] = _pad_rc(rf1b[...].reshape(1, DFF), 1, DFFP)
        f2w_s[...] = _bf(_pad_rc(rf2w[...], DFFP, FP))
        f2b_s[...] = _pad_rc(rf2b[...].reshape(1, F), 1, FP)
        n1a_s[...] = _pad_rc(rn1a[...].reshape(1, F), 1, FP)
        n1b_s[...] = _pad_rc(rn1b[...].reshape(1, F), 1, FP)
        n2a_s[...] = _pad_rc(rn2a[...].reshape(1, F), 1, FP)
        n2b_s[...] = _pad_rc(rn2b[...].reshape(1, F), 1, FP)
        pf_s[...] = _bf(_pad_rc(rpuw[...][:F, :], FP, P))
        pp_s[...] = _bf(rpuw[...][F:, :])
        s1w_s[...] = _bf(_pad_rc(rs1w[...], FP, HLFP))
        s1b_s[...] = _pad_rc(rs1b[...].reshape(1, HLF), 1, HLFP)
        s2w_s[...] = _pad_rc(rs2w[...].reshape(1, HLF), 1, HLFP)
        c2w_s[...] = _pad_rc(rc2w[...].reshape(1, HLF), 1, HLFP)
        c1w_s[...] = _bf(_pad_rc(rc1w[...], FP, HLFP))
        c1b_s[...] = _pad_rc(rc1b[...].reshape(1, HLF), 1, HLFP)

    s = fx_ref.shape[1]
    fx = fx_ref[...].reshape(BB * s, FP)
    px = px_ref[...].reshape(BB * s, P)
    fmask = (lax.broadcasted_iota(jnp.int32, (1, FP), 1) < F
             ).astype(jnp.float32)

    fxb = _bf(fx)
    q = jnp.dot(fxb, qw_s[...], preferred_element_type=jnp.float32) + qb_s[...]
    k = jnp.dot(fxb, kw_s[...], preferred_element_type=jnp.float32) + kb_s[...]
    v = jnp.dot(fxb, vw_s[...], preferred_element_type=jnp.float32) + vb_s[...]

    qb16, kb16, vb16 = _bf(q), _bf(k), _bf(v)
    scale = 1.0 / math.sqrt(DH)
    ctx_rows = []
    for i in range(BB):
        rs = slice(i * s, (i + 1) * s)
        ctxs = []
        for h in range(H):
            sl = slice(h * DHP, (h + 1) * DHP)
            sc = lax.dot_general(qb16[rs, sl], kb16[rs, sl],
                                 (((1,), (1,)), ((), ())),
                                 preferred_element_type=jnp.float32) * scale
            sc = sc - jnp.max(sc, axis=-1, keepdims=True)
            e = jnp.exp(sc)
            p = e / jnp.sum(e, axis=-1, keepdims=True)
            ctxs.append(jnp.dot(_bf(p), vb16[rs, sl],
                                preferred_element_type=jnp.float32))
        ctx_rows.append(jnp.concatenate(ctxs, axis=-1))
    ctx = _bf(jnp.concatenate(ctx_rows, axis=0))
    att = jnp.dot(ctx, ow_s[...], preferred_element_type=jnp.float32) + ob_s[...]

    fx1 = _ln(fx + att, n1a_s[...], n1b_s[...], fmask)

    ff = jnp.maximum(
        jnp.dot(_bf(fx1), f1w_s[...], preferred_element_type=jnp.float32)
        + f1b_s[...], 0.0)
    ff = jnp.dot(_bf(ff), f2w_s[...],
                 preferred_element_type=jnp.float32) + f2b_s[...]
    fx2 = _ln(fx1 + ff, n2a_s[...], n2b_s[...], fmask)

    fx2b = _bf(fx2)
    px_new = (jnp.dot(fx2b, pf_s[...], preferred_element_type=jnp.float32)
              + jnp.dot(_bf(px), pp_s[...], preferred_element_type=jnp.float32)
              + rpub[...].reshape(1, P))

    h1 = jnp.maximum(
        jnp.dot(fx2b, s1w_s[...], preferred_element_type=jnp.float32)
        + s1b_s[...], 0.0)
    sim = (jnp.sum(h1 * s2w_s[...], axis=-1, keepdims=True)
           + rs2b[...].reshape(1, 1))
    h2 = jnp.maximum(
        jnp.dot(fx2b, c1w_s[...], preferred_element_type=jnp.float32)
        + c1b_s[...], 0.0)
    conf = jax.nn.sigmoid(jnp.sum(h2 * c2w_s[...], axis=-1, keepdims=True)
                          + rc2b[...].reshape(1, 1))

    sm_rows = []
    for i in range(BB):
        rs = slice(i * s, (i + 1) * s)
        qrow = qrow_ref[i]
        qcol = qcol_ref[i]
        qmask = (qcol == qrow).astype(jnp.float32)
        m = sim[rs] * qmask
        m = m - jnp.max(m, axis=0, keepdims=True)
        e = jnp.exp(m)
        aw = e / jnp.sum(e, axis=0, keepdims=True)
        sm_rows.append(
            lax.dot_general(_bf(aw), _bf(px_new[rs]), (((0,), (0,)), ((), ())),
                            preferred_element_type=jnp.float32))
    smoothed = jnp.concatenate(sm_rows, axis=0)

    if fx_out is not None:
        fx_out[...] = fx2.reshape(BB, s, FP)
    px_out[...] = (conf * px_new
                   + (1.0 - conf) * smoothed).reshape(BB, s, P)


def _body_mid(*args):
    _layer_body(args, write_fx=True)


def _body_last(*args):
    _layer_body(args, write_fx=False)


def _whole(shape):
    nd = len(shape)
    return pl.BlockSpec(shape, lambda b, _nd=nd: (0,) * _nd)


_SCRATCH = [
    pltpu.VMEM((FP, FP), jnp.bfloat16),
    pltpu.VMEM((1, FP), jnp.float32),
    pltpu.VMEM((FP, FP), jnp.bfloat16),
    pltpu.VMEM((1, FP), jnp.float32),
    pltpu.VMEM((FP, FP), jnp.bfloat16),
    pltpu.VMEM((1, FP), jnp.float32),
    pltpu.VMEM((FP, FP), jnp.bfloat16),
    pltpu.VMEM((1, FP), jnp.float32),
    pltpu.VMEM((FP, DFFP), jnp.bfloat16),
    pltpu.VMEM((1, DFFP), jnp.float32),
    pltpu.VMEM((DFFP, FP), jnp.bfloat16),
    pltpu.VMEM((1, FP), jnp.float32),
    pltpu.VMEM((1, FP), jnp.float32),
    pltpu.VMEM((1, FP), jnp.float32),
    pltpu.VMEM((1, FP), jnp.float32),
    pltpu.VMEM((1, FP), jnp.float32),
    pltpu.VMEM((FP, P), jnp.bfloat16),
    pltpu.VMEM((P, P), jnp.bfloat16),
    pltpu.VMEM((FP, HLFP), jnp.bfloat16),
    pltpu.VMEM((1, HLFP), jnp.float32),
    pltpu.VMEM((1, HLFP), jnp.float32),
    pltpu.VMEM((FP, HLFP), jnp.bfloat16),
    pltpu.VMEM((1, HLFP), jnp.float32),
    pltpu.VMEM((1, HLFP), jnp.float32),
]


def _layer_call(fx, px, qrow, qcol, wts, last=False, interpret=False):
    b, s, _ = fx.shape
    in_specs = [
        pl.BlockSpec((BB, s, FP), lambda i: (i, 0, 0)),
        pl.BlockSpec((BB, s, P), lambda i: (i, 0, 0)),
        pl.BlockSpec((BB, 1, s), lambda i: (i, 0, 0)),
        pl.BlockSpec((BB, s, 1), lambda i: (i, 0, 0)),
    ] + [_whole(w.shape) for w in wts]
    px_spec = pl.BlockSpec((BB, s, P), lambda i: (i, 0, 0))
    px_shape = jax.ShapeDtypeStruct((b, s, P), jnp.float32)
    if last:
        out_specs, out_shape = px_spec, px_shape
        body = _body_last
    else:
        out_specs = [pl.BlockSpec((BB, s, FP), lambda i: (i, 0, 0)), px_spec]
        out_shape = [jax.ShapeDtypeStruct((b, s, FP), jnp.float32), px_shape]
        body = _body_mid
    out = pl.pallas_call(
        body,
        grid=(b // BB,),
        in_specs=in_specs,
        out_specs=out_specs,
        out_shape=out_shape,
        scratch_shapes=list(_SCRATCH),
        compiler_params=pltpu.CompilerParams(
            dimension_semantics=("arbitrary",),
        ),
        interpret=interpret,
    )(fx, px, qrow, qcol, *wts)
    if last:
        return None, out
    return out


def kernel(x, annotators, questions, embeddings, params):
    b, s = annotators.shape
    qidx = questions.astype(jnp.int32)
    ann = annotators.astype(jnp.int32)
    aidx = jnp.where(ann < 0, NA, ann)

    qt = jnp.pad(params["q_emb"], ((0, 0), (0, 128 - AED)))
    at = jnp.pad(params["a_emb"], ((0, 0), (0, 128 - AED)))
    qe, ae = _sc_gather(qt, at, qidx.reshape(-1), aidx.reshape(-1))
    emb = (qe + ae)[:, :AED].reshape(b, s, AED)

    fx = jnp.concatenate(
        [emb, embeddings, x[:, :, 1:],
         jnp.zeros((b, s, FP - F), jnp.float32)], axis=-1)
    px = x[:, :, 1:]
    qrow = qidx.reshape(b, 1, s)
    qcol = qidx.reshape(b, s, 1)

    for li, lp in enumerate(params["layers"]):
        wts = [lp[k][:, 0] if k in ("s2w", "c2w") else lp[k]
               for k in RAW_KEYS]
        fx, px = _layer_call(fx, px, qrow, qcol, wts,
                             last=(li == NLAYER - 1))
    return px
